# Initial kernel scaffold; baseline (speedup 1.0000x reference)
import jax
import jax.numpy as jnp
from jax.experimental import pallas as pl


def kernel(x, edge_index, adj_vals, W, prelu_a):
    raise NotImplementedError("write your pallas kernel here")


# speedup vs baseline: 4.0845x; 4.0845x over previous
import functools

import jax
import jax.numpy as jnp
from jax import lax
from jax.experimental import pallas as pl
from jax.experimental.pallas import tpu as pltpu
from jax.experimental.pallas import tpu_sc as plsc


def _make_sc_spmm(N, E, D, NC, NS):
  NW = NC * NS
  EW = E // NW
  C = 80
  NCHUNK = EW // C
  RPT = N // NS
  LANES = D // 16

  mesh = plsc.VectorSubcoreMesh(core_axis_name="c", subcore_axis_name="s")

  @functools.partial(
      pl.kernel,
      out_type=jax.ShapeDtypeStruct((NC, N, D), jnp.float32),
      mesh=mesh,
      scratch_types=[
          pltpu.VMEM((C,), jnp.int32),
          pltpu.VMEM((C,), jnp.int32),
          pltpu.VMEM((C,), jnp.float32),
          pltpu.VMEM((C, D), jnp.float32),
          pltpu.VMEM_SHARED((N, D), jnp.float32),
          pltpu.SemaphoreType.DMA,
      ],
      compiler_params=pltpu.CompilerParams(needs_layout_passes=False),
  )
  def sc_spmm(x_hbm, row_hbm, col_hbm, val_hbm, out_hbm,
              cidx, ridx, vals, rows, acc, sem):
    cid = lax.axis_index("c")
    sid = lax.axis_index("s")
    wid = cid * NS + sid

    def zrow(i, _):
      for j in range(LANES):
        rows[i, pl.ds(j * 16, 16)] = jnp.zeros((16,), jnp.float32)
      return 0
    lax.fori_loop(0, C, zrow, 0)
    n_copies = N // C
    n_rounds = (n_copies + NS - 1) // NS
    for m in range(n_rounds):
      idx = sid + NS * m
      @pl.when(idx < n_copies)
      def _():
        pltpu.sync_copy(rows, acc.at[pl.ds(pl.multiple_of(idx * C, 8), C)])
    plsc.subcore_barrier()

    def chunk(k, _):
      base = pl.multiple_of(wid * EW + k * C, 8)
      pltpu.sync_copy(col_hbm.at[pl.ds(base, C)], cidx)
      pltpu.sync_copy(row_hbm.at[pl.ds(base, C)], ridx)
      pltpu.sync_copy(val_hbm.at[pl.ds(base, C)], vals)
      pltpu.async_copy(x_hbm.at[cidx], rows, sem).wait()

      def srow(i, _):
        v = plsc.load_gather(vals, [jnp.zeros((16,), jnp.int32) + i])
        for j in range(LANES):
          sl = pl.ds(j * 16, 16)
          rows[i, sl] = rows[i, sl] * v
        return 0
      lax.fori_loop(0, C, srow, 0)

      pltpu.sync_copy(rows, acc.at[ridx], add=True)
      return 0
    lax.fori_loop(0, NCHUNK, chunk, 0)

    plsc.subcore_barrier()

    for m in range(n_rounds):
      idx = sid + NS * m
      @pl.when(idx < n_copies)
      def _():
        off = pl.multiple_of(idx * C, 8)
        pltpu.sync_copy(acc.at[pl.ds(off, C)],
                        out_hbm.at[cid, pl.ds(off, C)])

  return sc_spmm


def _tc_matmul_prelu(partials, W, prelu_a, N, D, NC):
  BR = 1000
  grid = (N // BR,)

  def body(a_ref, p_ref, w_ref, o_ref):
    s = p_ref[0]
    for c in range(1, NC):
      s = s + p_ref[c]
    h = jnp.dot(s, w_ref[...], preferred_element_type=jnp.float32)
    a = a_ref[0, 0]
    o_ref[...] = jnp.where(h >= 0, h, a * h)

  return pl.pallas_call(
      body,
      grid=grid,
      in_specs=[
          pl.BlockSpec((1, 1), lambda i: (0, 0)),
          pl.BlockSpec((NC, BR, D), lambda i: (0, i, 0)),
          pl.BlockSpec((D, D), lambda i: (0, 0)),
      ],
      out_specs=pl.BlockSpec((BR, D), lambda i: (i, 0)),
      out_shape=jax.ShapeDtypeStruct((N, D), jnp.float32),
  )(prelu_a.reshape(1, 1), partials, W)


def kernel(x, edge_index, adj_vals, W, prelu_a):
  N, D = x.shape
  E = adj_vals.shape[0]
  info = plsc.get_sparse_core_info()
  NC, NS = info.num_cores, info.num_subcores

  row = edge_index[0].astype(jnp.int32)
  col = edge_index[1].astype(jnp.int32)

  sc_spmm = _make_sc_spmm(N, E, D, NC, NS)
  partials = sc_spmm(x, row, col, adj_vals)
  return _tc_matmul_prelu(partials, W, prelu_a, N, D, NC)
